# baseline (speedup 1.0000x reference)
import functools

import jax
import jax.numpy as jnp
from jax import lax
from jax.experimental import pallas as pl
from jax.experimental.pallas import tpu as pltpu

_B, _N, _C = 2, 4096, 256
_NB = _B * _N
_NS = 16
_TEMP = 0.1
_RB1 = 512
_RB2 = 256
_INF = float("inf")
_HUGE = 1e30
_BIGI = 2**30


def _boundary_body(pos_ref, an_ref, lab_ref, qpos_ref, qan_ref, qlab_ref,
                   feat_ref, mask_ref, norm_ref, d_ref, key_ref):
    anrow = an_ref[0, 0, :].reshape(1, _N)
    qan = qan_ref[0, 0, 0, :].reshape(_RB1, 1)
    c = lax.dot_general(qpos_ref[0], pos_ref[0], (((1,), (0,)), ((), ())),
                        precision=lax.Precision.DEFAULT,
                        preferred_element_type=jnp.float32)
    d_ref[...] = qan + anrow - 2.0 * c

    labrow = lab_ref[0, 0, :].reshape(1, _N)
    qlab = qlab_ref[0, 0, 0, :]
    iota = lax.broadcasted_iota(jnp.int32, (_RB1, _N), 1)
    key_ref[...] = iota * 32 + labrow

    def body(t, cnt):
        d = d_ref[...]
        m = jnp.min(d, axis=1, keepdims=True)
        key = key_ref[...]
        jml = jnp.min(jnp.where(d == m, key, _BIGI), axis=1, keepdims=True)
        d_ref[...] = jnp.where(key == jml, _INF, d)
        labt = jml[:, 0] & 31
        inc = jnp.where((labt != qlab) & (t >= 1), 1, 0)
        return cnt + inc

    cnt_diff = lax.fori_loop(0, 9, body, jnp.zeros((_RB1,), jnp.int32))
    boundary = cnt_diff > 4
    mask_ref[...] = boundary.astype(jnp.int32).reshape(1, 1, _RB1)

    fq = feat_ref[0]
    nrm = jnp.sqrt(jnp.sum(fq * fq, axis=1))
    norm_ref[...] = nrm.reshape(1, 1, _RB1)


def _loss_body(pos_ref, an_ref, lab_ref, mask_ref, norm_ref,
               qpos_ref, qan_ref, qlab_ref, qmask_ref, qnorm_ref,
               featq_ref, fthi_ref, ftlo_ref, out_ref, d_ref, key_ref):
    maskrow = mask_ref[0, :].reshape(1, _NB)
    m_total = jnp.sum(mask_ref[0, :])
    k2 = jnp.minimum(_NS, m_total - 1)

    anrow = an_ref[0, :].reshape(1, _NB)
    qan = qan_ref[0, 0, :].reshape(_RB2, 1)
    c = lax.dot_general(qpos_ref[...], pos_ref[...], (((1,), (0,)), ((), ())),
                        precision=lax.Precision.DEFAULT,
                        preferred_element_type=jnp.float32)
    d = qan + anrow - 2.0 * c
    d_ref[...] = jnp.where(maskrow != 0, d, _HUGE)

    rn_all = 1.0 / jnp.clip(norm_ref[0, :], 1e-12, None).reshape(1, _NB)
    rn_q = 1.0 / jnp.clip(qnorm_ref[0, 0, :], 1e-12, None)
    fq = featq_ref[...] * (rn_q * jnp.float32(1.0 / _TEMP)).reshape(_RB2, 1)
    fq_hi = fq.astype(jnp.bfloat16)
    fq_lo = (fq - fq_hi.astype(jnp.float32)).astype(jnp.bfloat16)
    dn = (((1,), (0,)), ((), ()))
    sim = lax.dot_general(fq_hi, fthi_ref[...], dn,
                          preferred_element_type=jnp.float32)
    sim = sim + lax.dot_general(fq_hi, ftlo_ref[...], dn,
                                preferred_element_type=jnp.float32)
    sim = sim + lax.dot_general(fq_lo, fthi_ref[...], dn,
                                preferred_element_type=jnp.float32)
    sim = sim * rn_all

    labrow = lab_ref[0, :].reshape(1, _NB)
    qlab = qlab_ref[0, 0, :]
    iota = lax.broadcasted_iota(jnp.int32, (_RB2, _NB), 1)
    key_ref[...] = iota * 32 + labrow

    def body(t, _):
        dcur = d_ref[...]
        m = jnp.min(dcur, axis=1, keepdims=True)
        key = key_ref[...]
        jml = jnp.min(jnp.where(dcur == m, key, _BIGI),
                      axis=1, keepdims=True)
        take = t < k2
        d_ref[...] = jnp.where((key == jml) & take, _INF, dcur)
        return 0

    lax.fori_loop(0, _NS, body, 0)

    dfin = d_ref[...]
    inmask = dfin == _INF
    labeq = labrow == qlab.reshape(_RB2, 1)
    e = jnp.where(inmask, jnp.exp(sim), 0.0)
    pos_exp = jnp.sum(jnp.where(labeq, e, 0.0), axis=1)
    all_exp = jnp.sum(e, axis=1)
    has_neg = jnp.max(jnp.where(inmask & (~labeq), 1.0, 0.0), axis=1)
    has_pos = pos_exp

    qmask = qmask_ref[0, 0, :]
    valid = (qmask != 0) & (has_pos > 0.0) & (has_neg > 0.0)
    loss = -jnp.log(pos_exp / all_exp + 1e-8)
    total = jnp.sum(jnp.where(valid, loss, 0.0))
    cnt = jnp.sum(valid.astype(jnp.float32))
    lane = lax.broadcasted_iota(jnp.int32, (1, 1, 128), 2)
    out_ref[...] = jnp.where(lane == 0, total,
                             jnp.where(lane == 1, cnt, 0.0))


def kernel(features, positions, labels):
    an = jnp.sum(positions * positions, axis=2)
    posT = positions.transpose(0, 2, 1)
    an3 = an.reshape(_B, 1, _N)
    an4 = an.reshape(_B, _N // _RB1, 1, _RB1)
    lab3 = labels.reshape(_B, 1, _N)
    lab4 = labels.reshape(_B, _N // _RB1, 1, _RB1)

    mask, norms = pl.pallas_call(
        _boundary_body,
        grid=(_B, _N // _RB1),
        in_specs=[
            pl.BlockSpec((1, 3, _N), lambda b, r: (b, 0, 0)),
            pl.BlockSpec((1, 1, _N), lambda b, r: (b, 0, 0)),
            pl.BlockSpec((1, 1, _N), lambda b, r: (b, 0, 0)),
            pl.BlockSpec((1, _RB1, 3), lambda b, r: (b, r, 0)),
            pl.BlockSpec((1, 1, 1, _RB1), lambda b, r: (b, r, 0, 0)),
            pl.BlockSpec((1, 1, 1, _RB1), lambda b, r: (b, r, 0, 0)),
            pl.BlockSpec((1, _RB1, _C), lambda b, r: (b, r, 0)),
        ],
        out_specs=[
            pl.BlockSpec((1, 1, _RB1), lambda b, r: (b, 0, r)),
            pl.BlockSpec((1, 1, _RB1), lambda b, r: (b, 0, r)),
        ],
        out_shape=[
            jax.ShapeDtypeStruct((_B, 1, _N), jnp.int32),
            jax.ShapeDtypeStruct((_B, 1, _N), jnp.float32),
        ],
        scratch_shapes=[pltpu.VMEM((_RB1, _N), jnp.float32),
                        pltpu.VMEM((_RB1, _N), jnp.int32)],
        compiler_params=pltpu.CompilerParams(
            dimension_semantics=("arbitrary", "arbitrary")),
    )(posT, an3, lab3, positions, an4, lab4, features)

    nblocks = _NB // _RB2
    pos_flat = positions.reshape(_NB, 3)
    feat_flat = features.reshape(_NB, _C)
    featT = feat_flat.T
    ft_hi = featT.astype(jnp.bfloat16)
    ft_lo = (featT - ft_hi.astype(jnp.float32)).astype(jnp.bfloat16)

    out = pl.pallas_call(
        _loss_body,
        grid=(nblocks,),
        in_specs=[
            pl.BlockSpec((3, _NB), lambda i: (0, 0)),
            pl.BlockSpec((1, _NB), lambda i: (0, 0)),
            pl.BlockSpec((1, _NB), lambda i: (0, 0)),
            pl.BlockSpec((1, _NB), lambda i: (0, 0)),
            pl.BlockSpec((1, _NB), lambda i: (0, 0)),
            pl.BlockSpec((_RB2, 3), lambda i: (i, 0)),
            pl.BlockSpec((1, 1, _RB2), lambda i: (i, 0, 0)),
            pl.BlockSpec((1, 1, _RB2), lambda i: (i, 0, 0)),
            pl.BlockSpec((1, 1, _RB2), lambda i: (i, 0, 0)),
            pl.BlockSpec((1, 1, _RB2), lambda i: (i, 0, 0)),
            pl.BlockSpec((_RB2, _C), lambda i: (i, 0)),
            pl.BlockSpec((_C, _NB), lambda i: (0, 0)),
            pl.BlockSpec((_C, _NB), lambda i: (0, 0)),
        ],
        out_specs=pl.BlockSpec((1, 1, 128), lambda i: (i, 0, 0)),
        out_shape=jax.ShapeDtypeStruct((nblocks, 1, 128), jnp.float32),
        scratch_shapes=[pltpu.VMEM((_RB2, _NB), jnp.float32),
                        pltpu.VMEM((_RB2, _NB), jnp.int32)],
        compiler_params=pltpu.CompilerParams(
            dimension_semantics=("arbitrary",)),
    )(pos_flat.T, an.reshape(1, _NB), labels.reshape(1, _NB),
      mask.reshape(1, _NB), norms.reshape(1, _NB),
      pos_flat, an.reshape(nblocks, 1, _RB2), labels.reshape(nblocks, 1, _RB2),
      mask.reshape(nblocks, 1, _RB2), norms.reshape(nblocks, 1, _RB2),
      feat_flat, ft_hi, ft_lo)

    total = out[:, 0, 0].sum()
    cnt = out[:, 0, 1].sum()
    return jnp.where(cnt > 0, total / jnp.maximum(cnt, 1.0),
                     jnp.asarray(0.0, dtype=jnp.float32))


# speedup vs baseline: 1.0039x; 1.0039x over previous
import functools

import jax
import jax.numpy as jnp
from jax import lax
from jax.experimental import pallas as pl
from jax.experimental.pallas import tpu as pltpu

_B, _N, _C = 2, 4096, 256
_NB = _B * _N
_NS = 16
_TEMP = 0.1
_RB1 = 512
_RB2 = 256
_INF = float("inf")
_BIGI = 2**30


def _boundary_body(pos_ref, an_ref, lab_ref, qpos_ref, qan_ref, qlab_ref,
                   feat_ref, mask_ref, norm_ref, d_ref, key_ref):
    anrow = an_ref[0, 0, :].reshape(1, _N)
    qan = qan_ref[0, 0, 0, :].reshape(_RB1, 1)
    c = lax.dot_general(qpos_ref[0], pos_ref[0], (((1,), (0,)), ((), ())),
                        precision=lax.Precision.DEFAULT,
                        preferred_element_type=jnp.float32)
    d_ref[...] = qan + anrow - 2.0 * c

    labrow = lab_ref[0, 0, :].reshape(1, _N)
    qlab = qlab_ref[0, 0, 0, :]
    iota = lax.broadcasted_iota(jnp.int32, (_RB1, _N), 1)
    key_ref[...] = iota * 32 + labrow

    def body(t, cnt):
        d = d_ref[...]
        m = jnp.min(d, axis=1, keepdims=True)
        key = key_ref[...]
        jml = jnp.min(jnp.where(d == m, key, _BIGI), axis=1, keepdims=True)
        d_ref[...] = jnp.where(key == jml, _INF, d)
        labt = jml[:, 0] & 31
        inc = jnp.where((labt != qlab) & (t >= 1), 1, 0)
        return cnt + inc

    cnt_diff = lax.fori_loop(0, 9, body, jnp.zeros((_RB1,), jnp.int32))
    boundary = cnt_diff > 4
    mask_ref[...] = boundary.astype(jnp.int32).reshape(1, 1, _RB1)

    fq = feat_ref[0]
    nrm = jnp.sqrt(jnp.sum(fq * fq, axis=1))
    norm_ref[...] = nrm.reshape(1, 1, _RB1)


def _loss_body(pos_ref, an_ref, lab_ref, mask_ref, norm_ref,
               qpos_ref, qan_ref, qlab_ref, qmask_ref, qnorm_ref,
               featq_ref, fthi_ref, ftlo_ref, out_ref, d_ref, key_ref):
    maskrow = mask_ref[0, :].reshape(1, _NB)
    m_total = jnp.sum(mask_ref[0, :])
    k2 = jnp.minimum(_NS, m_total - 1)

    anrow = an_ref[0, :].reshape(1, _NB)
    qan = qan_ref[0, 0, :].reshape(_RB2, 1)
    c = lax.dot_general(qpos_ref[...], pos_ref[...], (((1,), (0,)), ((), ())),
                        precision=lax.Precision.DEFAULT,
                        preferred_element_type=jnp.float32)
    d = qan + anrow - 2.0 * c
    d_ref[...] = jnp.where(maskrow != 0, d, _INF)

    rn_all = 1.0 / jnp.clip(norm_ref[0, :], 1e-12, None).reshape(1, _NB)
    rn_q = 1.0 / jnp.clip(qnorm_ref[0, 0, :], 1e-12, None)
    fq = featq_ref[...] * (rn_q * jnp.float32(1.0 / _TEMP)).reshape(_RB2, 1)
    fq_hi = fq.astype(jnp.bfloat16)
    fq_lo = (fq - fq_hi.astype(jnp.float32)).astype(jnp.bfloat16)
    dn = (((1,), (0,)), ((), ()))
    sim = lax.dot_general(fq_hi, fthi_ref[...], dn,
                          preferred_element_type=jnp.float32)
    sim = sim + lax.dot_general(fq_hi, ftlo_ref[...], dn,
                                preferred_element_type=jnp.float32)
    sim = sim + lax.dot_general(fq_lo, fthi_ref[...], dn,
                                preferred_element_type=jnp.float32)
    sim = sim * rn_all

    labrow = lab_ref[0, :].reshape(1, _NB)
    qlab = qlab_ref[0, 0, :]
    iota = lax.broadcasted_iota(jnp.int32, (_RB2, _NB), 1)
    key_ref[...] = iota * 32 + labrow

    def body(t, carry):
        pos_exp, all_exp, has_pos, has_neg = carry
        dcur = d_ref[...]
        m = jnp.min(dcur, axis=1, keepdims=True)
        key = key_ref[...]
        jml = jnp.min(jnp.where(dcur == m, key, _BIGI),
                      axis=1, keepdims=True)
        onehot = key == jml
        simsel = jnp.sum(jnp.where(onehot, sim, 0.0), axis=1)
        d_ref[...] = jnp.where(onehot, _INF, dcur)
        labt = jml[:, 0] & 31
        sv = t < k2
        pm = labt == qlab
        e = jnp.where(sv, jnp.exp(simsel), 0.0)
        pos_exp = pos_exp + jnp.where(pm, e, 0.0)
        all_exp = all_exp + e
        has_pos = jnp.maximum(has_pos, jnp.where(pm & sv, 1.0, 0.0))
        has_neg = jnp.maximum(has_neg, jnp.where((~pm) & sv, 1.0, 0.0))
        return pos_exp, all_exp, has_pos, has_neg

    zf = jnp.zeros((_RB2,), jnp.float32)
    pos_exp, all_exp, has_pos, has_neg = lax.fori_loop(
        0, _NS, body, (zf, zf, zf, zf))

    qmask = qmask_ref[0, 0, :]
    valid = (qmask != 0) & (has_pos > 0.0) & (has_neg > 0.0)
    loss = -jnp.log(pos_exp / all_exp + 1e-8)
    total = jnp.sum(jnp.where(valid, loss, 0.0))
    cnt = jnp.sum(valid.astype(jnp.float32))
    lane = lax.broadcasted_iota(jnp.int32, (1, 1, 128), 2)
    out_ref[...] = jnp.where(lane == 0, total,
                             jnp.where(lane == 1, cnt, 0.0))


def kernel(features, positions, labels):
    an = jnp.sum(positions * positions, axis=2)
    posT = positions.transpose(0, 2, 1)
    an3 = an.reshape(_B, 1, _N)
    an4 = an.reshape(_B, _N // _RB1, 1, _RB1)
    lab3 = labels.reshape(_B, 1, _N)
    lab4 = labels.reshape(_B, _N // _RB1, 1, _RB1)

    mask, norms = pl.pallas_call(
        _boundary_body,
        grid=(_B, _N // _RB1),
        in_specs=[
            pl.BlockSpec((1, 3, _N), lambda b, r: (b, 0, 0)),
            pl.BlockSpec((1, 1, _N), lambda b, r: (b, 0, 0)),
            pl.BlockSpec((1, 1, _N), lambda b, r: (b, 0, 0)),
            pl.BlockSpec((1, _RB1, 3), lambda b, r: (b, r, 0)),
            pl.BlockSpec((1, 1, 1, _RB1), lambda b, r: (b, r, 0, 0)),
            pl.BlockSpec((1, 1, 1, _RB1), lambda b, r: (b, r, 0, 0)),
            pl.BlockSpec((1, _RB1, _C), lambda b, r: (b, r, 0)),
        ],
        out_specs=[
            pl.BlockSpec((1, 1, _RB1), lambda b, r: (b, 0, r)),
            pl.BlockSpec((1, 1, _RB1), lambda b, r: (b, 0, r)),
        ],
        out_shape=[
            jax.ShapeDtypeStruct((_B, 1, _N), jnp.int32),
            jax.ShapeDtypeStruct((_B, 1, _N), jnp.float32),
        ],
        scratch_shapes=[pltpu.VMEM((_RB1, _N), jnp.float32),
                        pltpu.VMEM((_RB1, _N), jnp.int32)],
        compiler_params=pltpu.CompilerParams(
            dimension_semantics=("arbitrary", "arbitrary")),
    )(posT, an3, lab3, positions, an4, lab4, features)

    nblocks = _NB // _RB2
    pos_flat = positions.reshape(_NB, 3)
    feat_flat = features.reshape(_NB, _C)
    featT = feat_flat.T
    ft_hi = featT.astype(jnp.bfloat16)
    ft_lo = (featT - ft_hi.astype(jnp.float32)).astype(jnp.bfloat16)

    out = pl.pallas_call(
        _loss_body,
        grid=(nblocks,),
        in_specs=[
            pl.BlockSpec((3, _NB), lambda i: (0, 0)),
            pl.BlockSpec((1, _NB), lambda i: (0, 0)),
            pl.BlockSpec((1, _NB), lambda i: (0, 0)),
            pl.BlockSpec((1, _NB), lambda i: (0, 0)),
            pl.BlockSpec((1, _NB), lambda i: (0, 0)),
            pl.BlockSpec((_RB2, 3), lambda i: (i, 0)),
            pl.BlockSpec((1, 1, _RB2), lambda i: (i, 0, 0)),
            pl.BlockSpec((1, 1, _RB2), lambda i: (i, 0, 0)),
            pl.BlockSpec((1, 1, _RB2), lambda i: (i, 0, 0)),
            pl.BlockSpec((1, 1, _RB2), lambda i: (i, 0, 0)),
            pl.BlockSpec((_RB2, _C), lambda i: (i, 0)),
            pl.BlockSpec((_C, _NB), lambda i: (0, 0)),
            pl.BlockSpec((_C, _NB), lambda i: (0, 0)),
        ],
        out_specs=pl.BlockSpec((1, 1, 128), lambda i: (i, 0, 0)),
        out_shape=jax.ShapeDtypeStruct((nblocks, 1, 128), jnp.float32),
        scratch_shapes=[pltpu.VMEM((_RB2, _NB), jnp.float32),
                        pltpu.VMEM((_RB2, _NB), jnp.int32)],
        compiler_params=pltpu.CompilerParams(
            dimension_semantics=("arbitrary",)),
    )(pos_flat.T, an.reshape(1, _NB), labels.reshape(1, _NB),
      mask.reshape(1, _NB), norms.reshape(1, _NB),
      pos_flat, an.reshape(nblocks, 1, _RB2), labels.reshape(nblocks, 1, _RB2),
      mask.reshape(nblocks, 1, _RB2), norms.reshape(nblocks, 1, _RB2),
      feat_flat, ft_hi, ft_lo)

    total = out[:, 0, 0].sum()
    cnt = out[:, 0, 1].sum()
    return jnp.where(cnt > 0, total / jnp.maximum(cnt, 1.0),
                     jnp.asarray(0.0, dtype=jnp.float32))


# speedup vs baseline: 1.0041x; 1.0002x over previous
import jax
import jax.numpy as jnp
from jax import lax
from jax.experimental import pallas as pl
from jax.experimental.pallas import tpu as pltpu

_B, _N, _C = 2, 4096, 256
_NB = _B * _N
_NS = 16
_TEMP = 0.1
_RB1 = 512
_RB2 = 256
_INF = float("inf")
_BIGI = 2**30


def _boundary_body(pos_ref, an_ref, lab_ref, qpos_ref, qan_ref, qlab_ref,
                   feat_ref, mask_ref, norm_ref, d_ref, key_ref):
    anrow = an_ref[0, 0, :].reshape(1, _N)
    qan = qan_ref[0, 0, 0, :].reshape(_RB1, 1)
    c = lax.dot_general(qpos_ref[0], pos_ref[0], (((1,), (0,)), ((), ())),
                        precision=lax.Precision.DEFAULT,
                        preferred_element_type=jnp.float32)
    d_ref[...] = qan + anrow - 2.0 * c

    labrow = lab_ref[0, 0, :].reshape(1, _N)
    qlab = qlab_ref[0, 0, 0, :]
    iota = lax.broadcasted_iota(jnp.int32, (_RB1, _N), 1)
    key_ref[...] = iota * 32 + labrow

    def body(t, cnt):
        d = d_ref[...]
        m = jnp.min(d, axis=1, keepdims=True)
        key = key_ref[...]
        jml = jnp.min(jnp.where(d == m, key, _BIGI), axis=1, keepdims=True)
        d_ref[...] = jnp.where(key == jml, _INF, d)
        labt = jml[:, 0] & 31
        inc = jnp.where((labt != qlab) & (t >= 1), 1, 0)
        return cnt + inc

    cnt_diff = lax.fori_loop(0, 9, body, jnp.zeros((_RB1,), jnp.int32))
    boundary = cnt_diff > 4
    mask_ref[...] = boundary.astype(jnp.int32).reshape(1, 1, _RB1)

    fq = feat_ref[0]
    nrm = jnp.sqrt(jnp.sum(fq * fq, axis=1))
    norm_ref[...] = nrm.reshape(1, 1, _RB1)


def _loss_body(pos_ref, an_ref, lab_ref, mask_ref, norm_ref,
               qpos_ref, qan_ref, qlab_ref, qmask_ref, qnorm_ref,
               featq_ref, fthi_ref, ftlo_ref, out_ref, d_ref, key_ref):
    maskrow = mask_ref[0, :].reshape(1, _NB)
    m_total = jnp.sum(mask_ref[0, :])
    k2 = jnp.minimum(_NS, m_total - 1)

    anrow = an_ref[0, :].reshape(1, _NB)
    qan = qan_ref[0, 0, :].reshape(_RB2, 1)
    c = lax.dot_general(qpos_ref[...], pos_ref[...], (((1,), (0,)), ((), ())),
                        precision=lax.Precision.DEFAULT,
                        preferred_element_type=jnp.float32)
    d = qan + anrow - 2.0 * c
    d_ref[...] = jnp.where(maskrow != 0, d, _INF)

    rn_all = 1.0 / jnp.clip(norm_ref[0, :], 1e-12, None).reshape(1, _NB)
    rn_q = 1.0 / jnp.clip(qnorm_ref[0, 0, :], 1e-12, None)
    fq = featq_ref[...] * (rn_q * jnp.float32(1.0 / _TEMP)).reshape(_RB2, 1)
    fq_hi = fq.astype(jnp.bfloat16)
    fq_lo = (fq - fq_hi.astype(jnp.float32)).astype(jnp.bfloat16)
    dn = (((1,), (0,)), ((), ()))
    sim = lax.dot_general(fq_hi, fthi_ref[...], dn,
                          preferred_element_type=jnp.float32)
    sim = sim + lax.dot_general(fq_hi, ftlo_ref[...], dn,
                                preferred_element_type=jnp.float32)
    sim = sim + lax.dot_general(fq_lo, fthi_ref[...], dn,
                                preferred_element_type=jnp.float32)
    sim = sim * rn_all

    labrow = lab_ref[0, :].reshape(1, _NB)
    qlab = qlab_ref[0, 0, :]
    iota = lax.broadcasted_iota(jnp.int32, (_RB2, _NB), 1)
    key_ref[...] = iota * 32 + labrow

    def body(t, carry):
        pos_exp, all_exp, has_pos, has_neg = carry
        dcur = d_ref[...]
        m = jnp.min(dcur, axis=1, keepdims=True)
        key = key_ref[...]
        jml = jnp.min(jnp.where(dcur == m, key, _BIGI),
                      axis=1, keepdims=True)
        onehot = key == jml
        simsel = jnp.sum(jnp.where(onehot, sim, 0.0), axis=1)
        d_ref[...] = jnp.where(onehot, _INF, dcur)
        labt = jml[:, 0] & 31
        sv = t < k2
        pm = labt == qlab
        e = jnp.where(sv, jnp.exp(simsel), 0.0)
        pos_exp = pos_exp + jnp.where(pm, e, 0.0)
        all_exp = all_exp + e
        has_pos = jnp.maximum(has_pos, jnp.where(pm & sv, 1.0, 0.0))
        has_neg = jnp.maximum(has_neg, jnp.where((~pm) & sv, 1.0, 0.0))
        return pos_exp, all_exp, has_pos, has_neg

    zf = jnp.zeros((_RB2,), jnp.float32)
    pos_exp, all_exp, has_pos, has_neg = lax.fori_loop(
        0, _NS, body, (zf, zf, zf, zf))

    qmask = qmask_ref[0, 0, :]
    valid = (qmask != 0) & (has_pos > 0.0) & (has_neg > 0.0)
    loss = -jnp.log(pos_exp / all_exp + 1e-8)
    total = jnp.sum(jnp.where(valid, loss, 0.0))
    cnt = jnp.sum(valid.astype(jnp.float32))
    lane = lax.broadcasted_iota(jnp.int32, (1, 1, 128), 2)
    out_ref[...] = jnp.where(lane == 0, total,
                             jnp.where(lane == 1, cnt, 0.0))


def kernel(features, positions, labels):
    an = jnp.sum(positions * positions, axis=2)
    posT = positions.transpose(0, 2, 1)
    an3 = an.reshape(_B, 1, _N)
    an4 = an.reshape(_B, _N // _RB1, 1, _RB1)
    lab3 = labels.reshape(_B, 1, _N)
    lab4 = labels.reshape(_B, _N // _RB1, 1, _RB1)

    mask, norms = pl.pallas_call(
        _boundary_body,
        grid=(_B, _N // _RB1),
        in_specs=[
            pl.BlockSpec((1, 3, _N), lambda b, r: (b, 0, 0)),
            pl.BlockSpec((1, 1, _N), lambda b, r: (b, 0, 0)),
            pl.BlockSpec((1, 1, _N), lambda b, r: (b, 0, 0)),
            pl.BlockSpec((1, _RB1, 3), lambda b, r: (b, r, 0)),
            pl.BlockSpec((1, 1, 1, _RB1), lambda b, r: (b, r, 0, 0)),
            pl.BlockSpec((1, 1, 1, _RB1), lambda b, r: (b, r, 0, 0)),
            pl.BlockSpec((1, _RB1, _C), lambda b, r: (b, r, 0)),
        ],
        out_specs=[
            pl.BlockSpec((1, 1, _RB1), lambda b, r: (b, 0, r)),
            pl.BlockSpec((1, 1, _RB1), lambda b, r: (b, 0, r)),
        ],
        out_shape=[
            jax.ShapeDtypeStruct((_B, 1, _N), jnp.int32),
            jax.ShapeDtypeStruct((_B, 1, _N), jnp.float32),
        ],
        scratch_shapes=[pltpu.VMEM((_RB1, _N), jnp.float32),
                        pltpu.VMEM((_RB1, _N), jnp.int32)],
        compiler_params=pltpu.CompilerParams(
            dimension_semantics=("arbitrary", "arbitrary")),
    )(posT, an3, lab3, positions, an4, lab4, features)

    nblocks = _NB // _RB2
    pos_flat = positions.reshape(_NB, 3)
    feat_flat = features.reshape(_NB, _C)
    featT = feat_flat.T
    ft_hi = featT.astype(jnp.bfloat16)
    ft_lo = (featT - ft_hi.astype(jnp.float32)).astype(jnp.bfloat16)

    out = pl.pallas_call(
        _loss_body,
        grid=(nblocks,),
        in_specs=[
            pl.BlockSpec((3, _NB), lambda i: (0, 0)),
            pl.BlockSpec((1, _NB), lambda i: (0, 0)),
            pl.BlockSpec((1, _NB), lambda i: (0, 0)),
            pl.BlockSpec((1, _NB), lambda i: (0, 0)),
            pl.BlockSpec((1, _NB), lambda i: (0, 0)),
            pl.BlockSpec((_RB2, 3), lambda i: (i, 0)),
            pl.BlockSpec((1, 1, _RB2), lambda i: (i, 0, 0)),
            pl.BlockSpec((1, 1, _RB2), lambda i: (i, 0, 0)),
            pl.BlockSpec((1, 1, _RB2), lambda i: (i, 0, 0)),
            pl.BlockSpec((1, 1, _RB2), lambda i: (i, 0, 0)),
            pl.BlockSpec((_RB2, _C), lambda i: (i, 0)),
            pl.BlockSpec((_C, _NB), lambda i: (0, 0)),
            pl.BlockSpec((_C, _NB), lambda i: (0, 0)),
        ],
        out_specs=pl.BlockSpec((1, 1, 128), lambda i: (i, 0, 0)),
        out_shape=jax.ShapeDtypeStruct((nblocks, 1, 128), jnp.float32),
        scratch_shapes=[pltpu.VMEM((_RB2, _NB), jnp.float32),
                        pltpu.VMEM((_RB2, _NB), jnp.int32)],
        compiler_params=pltpu.CompilerParams(
            dimension_semantics=("arbitrary",)),
    )(pos_flat.T, an.reshape(1, _NB), labels.reshape(1, _NB),
      mask.reshape(1, _NB), norms.reshape(1, _NB),
      pos_flat, an.reshape(nblocks, 1, _RB2), labels.reshape(nblocks, 1, _RB2),
      mask.reshape(nblocks, 1, _RB2), norms.reshape(nblocks, 1, _RB2),
      feat_flat, ft_hi, ft_lo)

    total = out[:, 0, 0].sum()
    cnt = out[:, 0, 1].sum()
    return jnp.where(cnt > 0, total / jnp.maximum(cnt, 1.0),
                     jnp.asarray(0.0, dtype=jnp.float32))
